# baseline (speedup 1.0000x reference)
import jax
import jax.numpy as jnp
from jax import lax
from jax.experimental import pallas as pl
from jax.experimental.pallas import tpu as pltpu

VTH_ = 1.0
T_ = 8

F32 = jnp.float32
BF16 = jnp.bfloat16

S0, O0, M0, R0 = 16, 32, 256, 320
S1, O1, M1, R1 = 16, 32, 128, 192
SI, MI = 66, 2112
RH = 4356
OH = 72
REO = 2184
SH, OHN, MH, RHN = 24, 32, 384, 448


def _rowmask(m, s, v, c):
    rf = lax.broadcasted_iota(jnp.int32, (m, c), 0).astype(F32)
    pj = rf - jnp.floor((rf + 0.5) * (1.0 / s)) * s
    return pj < v


def _aff_clip(acc, af_ref):
    return jnp.clip(acc * af_ref[0:1, :] + af_ref[1:2, :], 0.0, VTH_)


def _conv1_phase(srcc, w_ref, a, b, s, o, m):
    acc = None
    for di in range(3):
        qa = a + di - 1
        pa, du = qa & 1, (qa - (qa & 1)) // 2
        for dj in range(3):
            qb = b + dj - 1
            pb, dv = qb & 1, (qb - (qb & 1)) // 2
            st = o + s * du
            d = jnp.dot(srcc[dv + 1, pa * 2 + pb, st:st + m, :],
                        w_ref[di * 3 + dj], preferred_element_type=F32)
            acc = d if acc is None else acc + d
    return acc


def _conv2_full(srcc, w_ref, s, o, m):
    acc = None
    for di in range(3):
        q = di - 1
        pa, du = q & 1, (q - (q & 1)) // 2
        for dj in range(3):
            q2 = dj - 1
            pb, dv = q2 & 1, (q2 - (q2 & 1)) // 2
            st = o + s * du
            d = jnp.dot(srcc[dv + 1, pa * 2 + pb, st:st + m, :],
                        w_ref[di * 3 + dj], preferred_element_type=F32)
            acc = d if acc is None else acc + d
    return acc


def _convt_phase(s2c, w_ref, e, f, s, o, m):
    acc = None
    for di in range(3):
        if (di & 1) == (e & 1):
            continue
        du = (e + di - 1) // 2
        for dj in range(3):
            if (dj & 1) == (f & 1):
                continue
            dv = (f + dj - 1) // 2
            st = o + s * du
            d = jnp.dot(s2c[dv + 1, st:st + m, :], w_ref[di * 3 + dj],
                        preferred_element_type=F32)
            acc = d if acc is None else acc + d
    return acc


def _store3s(ref, ph, o, m, s, valf, vmod, valid=None, copies=(-1, 0, 1)):
    c = valf.shape[1]
    if valid is not None:
        valf = jnp.where(valid, valf, 0.0)
    vb = valf.astype(BF16)
    zrow = jnp.zeros((1, c), BF16)
    zb = jnp.zeros((), BF16)
    for cv in copies:
        if cv == -1:
            dn = jnp.concatenate([zrow, vb[:m - 1, :]], 0)
            cval = jnp.where(vmod == 0, zb, dn)
        elif cv == 1:
            up = jnp.concatenate([vb[1:, :], zrow], 0)
            cval = jnp.where(vmod == s - 1, zb, up)
        else:
            cval = vb
        if ph is None:
            ref[cv + 1, o:o + m, :] = cval
        else:
            ref[cv + 1, ph, o:o + m, :] = cval


def _main_kernel(x9, w27, af_di1, wdi2, af_di2, wwx, af_wx, wwxn, af_wxn,
                 ws1, af_s1, ws2, af_s2, ws1n, af_s1n, ws2n, af_s2n,
                 wtr, wtrn, wds, af_ds, winc, af_inc, out,
                 E, Ob, H2, HN, A0, S1b, S2b, I0, A1, S1n, S2n, I1):
    m66 = _rowmask(MI, SI, 32, 64)
    m24 = _rowmask(MH, SH, 16, 64)
    zf = jnp.float32(0.0)

    def vmod16(m, c):
        return lax.broadcasted_iota(jnp.int32, (m, c), 0) & 15

    vm0_64 = vmod16(M0, 64)
    vm0_128 = vmod16(M0, 128)
    vm0_256 = vmod16(M0, 256)
    vm1_64 = vmod16(M1, 64)
    vm1_128 = vmod16(M1, 128)
    vm1_256 = vmod16(M1, 256)
    val1_64 = vm1_64 < 8
    val1_128 = vm1_128 < 8
    val1_256 = vm1_256 < 8

    for rr in (A0, S1b, A1, S1n, S2b, S2n):
        rr[...] = jnp.zeros_like(rr)
    HN[...] = jnp.zeros_like(HN)
    H2[0:OH, :] = jnp.zeros((OH, 64), F32)
    H2[OH + MI:RH, :] = jnp.zeros((RH - OH - MI, 64), F32)
    E[2176:REO, :] = jnp.zeros((REO - 2176, 64), F32)
    Ob[2176:REO, :] = jnp.zeros((REO - 2176, 64), F32)

    rf = lax.broadcasted_iota(jnp.int32, (RH, 64), 0).astype(F32)
    pj = rf - jnp.floor((rf + 0.5) * (1.0 / SI)) * SI
    mint = (rf >= SI) & (rf < RH - SI) & (pj >= 1) & (pj < 65)
    h1 = jnp.dot(x9[0], w27[...], preferred_element_type=F32)
    h1 = jnp.where(mint, _aff_clip(h1, af_di1), zf)
    v3 = h1.reshape(2178, 2, 64)
    E[0:2178, :] = v3[:, 0, :]
    Ob[0:2178, :] = v3[:, 1, :]

    acc = None
    for di in range(3):
        for dj in range(3):
            off = di * SI + dj
            src = Ob if (off & 1) else E
            b0 = off // 2
            d = jnp.dot(src[b0:b0 + MI, :], wdi2[di * 3 + dj],
                        preferred_element_type=F32)
            acc = d if acc is None else acc + d
    hv = jnp.where(m66, _aff_clip(acc, af_di2), zf)
    H2[OH:OH + MI, :] = hv

    h4 = hv.reshape(32, 33, 2, 64)
    hm = (h4[:, :, 0, :] + h4[:, :, 1, :]) * 0.5
    hm2 = hm.reshape(16, 2, 33, 64)
    hm3 = (hm2[:, 0, :, :] + hm2[:, 1, :, :]) * 0.5
    hn24 = jnp.concatenate([hm3[:, 0:16, :], jnp.zeros((16, 8, 64), F32)], 1)
    HN[OHN:OHN + MH, :] = hn24.reshape(MH, 64)

    acc = None
    for di in range(3):
        for dj in range(3):
            st = OH + SI * (di - 1) + (dj - 1)
            d = jnp.dot(H2[st:st + MI, :], wwx[di * 3 + dj],
                        preferred_element_type=F32)
            acc = d if acc is None else acc + d
    i0v = jnp.where(m66, _aff_clip(acc, af_wx), zf)
    i4 = i0v.reshape(32, 33, 2, 64)
    for b in range(2):
        i5 = i4[:, :, b, :].reshape(16, 2, 33, 64)
        for a in range(2):
            I0[a * 2 + b, :, :] = i5[:, a, 0:16, :].reshape(M0, 64)

    acc = None
    for di in range(3):
        for dj in range(3):
            st = OHN + SH * (di - 1) + (dj - 1)
            d = jnp.dot(HN[st:st + MH, :], wwxn[di * 3 + dj],
                        preferred_element_type=F32)
            acc = d if acc is None else acc + d
    i1v = jnp.where(m24, _aff_clip(acc, af_wxn), zf)
    i6 = i1v.reshape(16, 12, 2, 64)
    for b in range(2):
        i7 = i6[:, :, b, :].reshape(8, 2, 12, 64)
        for a in range(2):
            v = jnp.concatenate(
                [i7[:, a, :, :], jnp.zeros((8, 4, 64), F32)], 1)
            I1[a * 2 + b, :, :] = v.reshape(M1, 64)

    def half_step0():
        for a in range(2):
            for b in range(2):
                acc = _conv1_phase(A0, ws1, a, b, S0, O0, M0)
                _store3s(S1b, a * 2 + b, O0, M0, S0,
                         _aff_clip(acc, af_s1), vm0_128, copies=(-1, 0))
        acc = _conv2_full(S1b, ws2, S0, O0, M0)
        _store3s(S2b, None, O0, M0, S0, _aff_clip(acc, af_s2), vm0_256)

    def half_step1():
        for a in range(2):
            for b in range(2):
                acc = _conv1_phase(A1, ws1n, a, b, S1, O1, M1)
                _store3s(S1n, a * 2 + b, O1, M1, S1,
                         _aff_clip(acc, af_s1n), vm1_128, valid=val1_128,
                         copies=(-1, 0))
        acc = _conv2_full(S1n, ws2n, S1, O1, M1)
        _store3s(S2n, None, O1, M1, S1, _aff_clip(acc, af_s2n), vm1_256,
                 valid=val1_256, copies=(0, 1))

    def step(_, carry):
        half_step0()
        half_step1()
        for e in range(2):
            for f in range(2):
                t0 = _convt_phase(S2b, wtr, e, f, S0, O0, M0)
                _store3s(A0, e * 2 + f, O0, M0, S0,
                         jnp.clip(t0 + I0[e * 2 + f, :, :], 0.0, VTH_),
                         vm0_64)
                t1 = _convt_phase(S2n, wtrn, e, f, S1, O1, M1)
                _store3s(A1, e * 2 + f, O1, M1, S1,
                         jnp.clip(t1 + I1[e * 2 + f, :, :], 0.0, VTH_),
                         vm1_64, valid=val1_64)
        return carry

    lax.fori_loop(0, T_, step, 0)

    half_step0()
    half_step1()

    s2c0 = S2b[0, :, :].reshape(R0 // 2, 2, 256)
    s2c1 = S2b[1, :, :].reshape(R0 // 2, 2, 256)
    e_c0 = s2c0[:, 0, :]
    e_c1, o_c1 = s2c1[:, 0, :], s2c1[:, 1, :]
    acc = None
    for di in range(3):
        for dj in range(3):
            src = (e_c0, e_c1, o_c1)[dj]
            b0 = 16 + 8 * (di - 1)
            d = jnp.dot(src[b0:b0 + M1, :], wds[di * 3 + dj],
                        preferred_element_type=F32)
            acc = d if acc is None else acc + d
    dsv = _aff_clip(acc, af_ds)
    dsc = dsv.reshape(8, 16, 256)[:, 0:8, :].reshape(64, 256)
    z1c = S2n[1, O1:O1 + M1, :].astype(F32).reshape(8, 16, 256)
    z1c = z1c[:, 0:8, :].reshape(64, 256)
    z = dsc + z1c
    zq = _aff_clip(jnp.dot(z, winc[...], preferred_element_type=F32), af_inc)
    out[0, :, :] = zq


def _cls_kernel(z, w, bias, out):
    out[...] = jnp.dot(z[...], w[...], preferred_element_type=F32) + bias[...]


def _tap9(w):
    return jnp.transpose(w, (2, 3, 1, 0)).reshape(9, w.shape[1], w.shape[0])


def _aff(p, bkey, bnkey):
    g = p[bnkey]['g']
    bb = p[bnkey]['b']
    bias = p[bkey] if bkey is not None else jnp.zeros_like(bb)
    return jnp.stack([g, bias * g + bb])


def kernel(x, params):
    p = params
    B = x.shape[0]

    xp = jnp.pad(x, ((0, 0), (0, 0), (1, 1), (1, 1)))
    pats = jnp.stack([xp[:, :, di:di + 64, dj:dj + 64]
                      for di in range(3) for dj in range(3)], axis=1)
    pats = pats.transpose(0, 3, 4, 1, 2).reshape(B, 64, 64, 27)
    pats = jnp.pad(pats, ((0, 0), (1, 1), (1, 1), (0, 5)))
    x9 = pats.reshape(B, RH, 32)

    w27 = jnp.transpose(p['di_w1'], (2, 3, 1, 0)).reshape(27, 64)
    w27 = jnp.concatenate([w27, jnp.zeros((5, 64), F32)], 0)

    def tr9(w):
        wf = jnp.flip(w, (2, 3)).transpose(1, 0, 2, 3)
        return _tap9(wf)

    weights = [
        w27, _aff(p, 'di_b1', 'di_bn1'),
        _tap9(p['di_w2']), _aff(p, 'di_b2', 'di_bn2'),
        _tap9(p['wx_w']), _aff(p, 'wx_b', 'wx_bn'),
        _tap9(p['wxn_w']), _aff(p, 'wxn_b', 'wxn_bn'),
        _tap9(p['s1_w']).astype(BF16), _aff(p, 's1_b', 's1_bn'),
        _tap9(p['s2_w']).astype(BF16), _aff(p, 's2_b', 's2_bn'),
        _tap9(p['s1n_w']).astype(BF16), _aff(p, 's1n_b', 's1n_bn'),
        _tap9(p['s2n_w']).astype(BF16), _aff(p, 's2n_b', 's2n_bn'),
        tr9(p['tr_w']).astype(BF16), tr9(p['trn_w']).astype(BF16),
        _tap9(p['ds_w']).astype(BF16), _aff(p, 'ds_b', 'ds_bn'),
        p['inc_w'][:, :, 0, 0].T, _aff(p, None, 'inc_bn'),
    ]

    scratch = [
        pltpu.VMEM((REO, 64), F32),
        pltpu.VMEM((REO, 64), F32),
        pltpu.VMEM((RH, 64), F32),
        pltpu.VMEM((RHN, 64), F32),
        pltpu.VMEM((3, 4, R0, 64), BF16),
        pltpu.VMEM((3, 4, R0, 128), BF16),
        pltpu.VMEM((3, R0, 256), BF16),
        pltpu.VMEM((4, M0, 64), F32),
        pltpu.VMEM((3, 4, R1, 64), BF16),
        pltpu.VMEM((3, 4, R1, 128), BF16),
        pltpu.VMEM((3, R1, 256), BF16),
        pltpu.VMEM((4, M1, 64), F32),
    ]
    half = B // 2
    wspecs = [pl.BlockSpec(w.shape, lambda c, i, nd=w.ndim: (0,) * nd)
              for w in weights]
    zmap = pl.pallas_call(
        _main_kernel,
        grid=(2, half),
        in_specs=[pl.BlockSpec((1, RH, 32),
                               lambda c, i: (c * half + i, 0, 0))] + wspecs,
        out_specs=pl.BlockSpec((1, 64, 256), lambda c, i: (c * half + i, 0, 0)),
        out_shape=jax.ShapeDtypeStruct((B, 64, 256), F32),
        scratch_shapes=scratch,
        compiler_params=pltpu.CompilerParams(
            dimension_semantics=("parallel", "arbitrary"),
            vmem_limit_bytes=100 * 1024 * 1024,
        ),
    )(x9, *weights)

    zflat = zmap.reshape(B, 64 * 256)
    wc = p['cls_w'].reshape(100, 256, 64).transpose(2, 1, 0).reshape(16384, 100)
    logits = pl.pallas_call(
        _cls_kernel,
        grid=(2,),
        in_specs=[
            pl.BlockSpec((B // 2, 16384), lambda i: (i, 0)),
            pl.BlockSpec((16384, 100), lambda i: (0, 0)),
            pl.BlockSpec((1, 100), lambda i: (0, 0)),
        ],
        out_specs=pl.BlockSpec((B // 2, 100), lambda i: (i, 0)),
        out_shape=jax.ShapeDtypeStruct((B, 100), F32),
        compiler_params=pltpu.CompilerParams(
            dimension_semantics=("parallel",),
        ),
    )(zflat, wc, p['cls_b'][None, :])
    return logits


# speedup vs baseline: 1.2156x; 1.2156x over previous
import jax
import jax.numpy as jnp
from jax import lax
from jax.experimental import pallas as pl
from jax.experimental.pallas import tpu as pltpu

VTH_ = 1.0
T_ = 8

F32 = jnp.float32
BF16 = jnp.bfloat16

S0, O0, M0, R0 = 16, 32, 256, 320
S1, O1, M1, R1 = 16, 32, 128, 192
SI, MI = 66, 2112
RH = 4356
OH = 72
REO = 2184
SH, OHN, MH, RHN = 24, 32, 384, 448


def _rowmask(m, s, v, c):
    rf = lax.broadcasted_iota(jnp.int32, (m, c), 0).astype(F32)
    pj = rf - jnp.floor((rf + 0.5) * (1.0 / s)) * s
    return pj < v


def _aff_clip(acc, af_ref):
    return jnp.clip(acc * af_ref[0:1, :] + af_ref[1:2, :], 0.0, VTH_)


def _shiftv(vb, cv, vmod, s):
    if cv == 0:
        return vb
    m, c = vb.shape
    zb = jnp.zeros((), BF16)
    if cv == -1:
        dn = jnp.concatenate([jnp.zeros((1, c), BF16), vb[:m - 1, :]], 0)
        return jnp.where(vmod == 0, zb, dn)
    up = jnp.concatenate([vb[1:, :], jnp.zeros((1, c), BF16)], 0)
    return jnp.where(vmod == s - 1, zb, up)


def _main_kernel(x9, w27, af_di1, wdi2, af_di2, wwx, af_wx, wwxn, af_wxn,
                 ws1, af_s1, ws2, af_s2, ws1n, af_s1n, ws2n, af_s2n,
                 wt00, wt01, wt10, wt11, wn00, wn01, wn10, wn11,
                 wds, af_ds, winc, af_inc, out,
                 E, Ob, H2, HN, A0S, S1T, S2U, S2M, A1S, S1nT, S2nU, I0, I1):
    m66 = _rowmask(MI, SI, 32, 64)
    m24 = _rowmask(MH, SH, 16, 64)
    zf = jnp.float32(0.0)

    def vmod16(m, c):
        return lax.broadcasted_iota(jnp.int32, (m, c), 0) & 15

    vm0_64 = vmod16(M0, 64)
    vm0_128 = vmod16(M0, 128)
    vm0_256 = vmod16(M0, 256)
    vm1_64 = vmod16(M1, 64)
    vm1_128 = vmod16(M1, 128)
    vm1_256 = vmod16(M1, 256)
    val1_64 = vm1_64 < 8
    val1_128 = vm1_128 < 8
    val1_256 = vm1_256 < 8

    for rr in (A0S, S1T, S2U, S2M, A1S, S1nT, S2nU):
        rr[...] = jnp.zeros_like(rr)
    HN[...] = jnp.zeros_like(HN)
    H2[0:OH, :] = jnp.zeros((OH, 64), F32)
    H2[OH + MI:RH, :] = jnp.zeros((RH - OH - MI, 64), F32)
    E[2176:REO, :] = jnp.zeros((REO - 2176, 64), F32)
    Ob[2176:REO, :] = jnp.zeros((REO - 2176, 64), F32)

    rf = lax.broadcasted_iota(jnp.int32, (RH, 64), 0).astype(F32)
    pj = rf - jnp.floor((rf + 0.5) * (1.0 / SI)) * SI
    mint = (rf >= SI) & (rf < RH - SI) & (pj >= 1) & (pj < 65)
    h1 = jnp.dot(x9[0], w27[...], preferred_element_type=F32)
    h1 = jnp.where(mint, _aff_clip(h1, af_di1), zf)
    v3 = h1.reshape(2178, 2, 64)
    E[0:2178, :] = v3[:, 0, :]
    Ob[0:2178, :] = v3[:, 1, :]

    acc = None
    for di in range(3):
        for dj in range(3):
            off = di * SI + dj
            src = Ob if (off & 1) else E
            b0 = off // 2
            d = jnp.dot(src[b0:b0 + MI, :], wdi2[di * 3 + dj],
                        preferred_element_type=F32)
            acc = d if acc is None else acc + d
    hv = jnp.where(m66, _aff_clip(acc, af_di2), zf)
    H2[OH:OH + MI, :] = hv

    h4 = hv.reshape(32, 33, 2, 64)
    hm = (h4[:, :, 0, :] + h4[:, :, 1, :]) * 0.5
    hm2 = hm.reshape(16, 2, 33, 64)
    hm3 = (hm2[:, 0, :, :] + hm2[:, 1, :, :]) * 0.5
    hn24 = jnp.concatenate([hm3[:, 0:16, :], jnp.zeros((16, 8, 64), F32)], 1)
    HN[OHN:OHN + MH, :] = hn24.reshape(MH, 64)

    acc = None
    for di in range(3):
        for dj in range(3):
            st = OH + SI * (di - 1) + (dj - 1)
            d = jnp.dot(H2[st:st + MI, :], wwx[di * 3 + dj],
                        preferred_element_type=F32)
            acc = d if acc is None else acc + d
    i0v = jnp.where(m66, _aff_clip(acc, af_wx), zf)
    i4 = i0v.reshape(32, 33, 2, 64)
    for b in range(2):
        i5 = i4[:, :, b, :].reshape(16, 2, 33, 64)
        for a in range(2):
            I0[a * 2 + b, :, :] = i5[:, a, 0:16, :].reshape(M0, 64)

    acc = None
    for di in range(3):
        for dj in range(3):
            st = OHN + SH * (di - 1) + (dj - 1)
            d = jnp.dot(HN[st:st + MH, :], wwxn[di * 3 + dj],
                        preferred_element_type=F32)
            acc = d if acc is None else acc + d
    i1v = jnp.where(m24, _aff_clip(acc, af_wxn), zf)
    i6 = i1v.reshape(16, 12, 2, 64)
    for b in range(2):
        i7 = i6[:, :, b, :].reshape(8, 2, 12, 64)
        for a in range(2):
            v = jnp.concatenate(
                [i7[:, a, :, :], jnp.zeros((8, 4, 64), F32)], 1)
            I1[a * 2 + b, :, :] = v.reshape(M1, 64)

    def asm_a(ref, av, o, m, vmod):
        for pa in range(2):
            for b in range(2):
                pieces = []
                for dj in range(3):
                    qb = b + dj - 1
                    pb, dv = qb & 1, (qb - (qb & 1)) // 2
                    pieces.append(_shiftv(av[(pa, pb)], dv, vmod, 16))
                ref[pa * 2 + b, o:o + m, :] = jnp.concatenate(pieces, 1)

    def conv1(aref, w3, a, b, s, o, m):
        acc = None
        for di in range(3):
            qa = a + di - 1
            pa, du = qa & 1, (qa - (qa & 1)) // 2
            st = o + s * du
            d = jnp.dot(aref[pa * 2 + b, st:st + m, :], w3[di],
                        preferred_element_type=F32)
            acc = d if acc is None else acc + d
        return acc

    def conv2(tref, w3, s, o, m):
        acc = None
        for di in range(3):
            q = di - 1
            pa, du = q & 1, (q - (q & 1)) // 2
            st = o + s * du
            d = jnp.dot(tref[pa, st:st + m, :], w3[di],
                        preferred_element_type=F32)
            acc = d if acc is None else acc + d
        return acc

    def half_step(aref, tref, uref, mref, w1, af1, w2, af2,
                  o, m, vmc1, vmc2, v1, v2):
        s1v = {}
        for a in range(2):
            for b in range(2):
                acc = conv1(aref, w1, a, b, 16, o, m)
                vv = _aff_clip(acc, af1)
                if v1 is not None:
                    vv = jnp.where(v1, vv, zf)
                s1v[(a, b)] = vv.astype(BF16)
        for a in range(2):
            v0, v1b = s1v[(a, 0)], s1v[(a, 1)]
            tref[a, o:o + m, :] = jnp.concatenate(
                [_shiftv(v1b, -1, vmc1, 16), v0, v1b], 1)
        acc = conv2(tref, w2, 16, o, m)
        vv = _aff_clip(acc, af2)
        if v2 is not None:
            vv = jnp.where(v2, vv, zf)
        s2b = vv.astype(BF16)
        uref[o:o + m, :] = jnp.concatenate(
            [s2b, _shiftv(s2b, 1, vmc2, 16)], 1)
        if mref is not None:
            mref[o:o + m, :] = _shiftv(s2b, -1, vmc2, 16)

    def half_step0():
        half_step(A0S, S1T, S2U, S2M, ws1, af_s1, ws2, af_s2,
                  O0, M0, vm0_128, vm0_256, None, None)

    def half_step1():
        half_step(A1S, S1nT, S2nU, None, ws1n, af_s1n, ws2n, af_s2n,
                  O1, M1, vm1_128, vm1_256, val1_128, val1_256)

    def convt(uref, wp00, wp01, wp10, wp11, iref, o, m, vmod, valid):
        av = {}
        c = uref.shape[-1] // 2
        t00 = jnp.dot(uref[o:o + m, 0:c], wp00[...],
                      preferred_element_type=F32)
        t01 = jnp.dot(uref[o:o + m, :], wp01[...],
                      preferred_element_type=F32)
        l10 = jnp.concatenate(
            [uref[o:o + m, 0:c], uref[o + 16:o + 16 + m, 0:c]], 1)
        t10 = jnp.dot(l10, wp10[...], preferred_element_type=F32)
        l11 = jnp.concatenate(
            [uref[o:o + m, :], uref[o + 16:o + 16 + m, :]], 1)
        t11 = jnp.dot(l11, wp11[...], preferred_element_type=F32)
        for (e, f), t in (((0, 0), t00), ((0, 1), t01),
                          ((1, 0), t10), ((1, 1), t11)):
            vv = jnp.clip(t + iref[e * 2 + f, :, :], 0.0, VTH_)
            if valid is not None:
                vv = jnp.where(valid, vv, zf)
            av[(e, f)] = vv.astype(BF16)
        return av

    def step(_, carry):
        half_step0()
        half_step1()
        av0 = convt(S2U, wt00, wt01, wt10, wt11, I0, O0, M0, vm0_64, None)
        asm_a(A0S, av0, O0, M0, vm0_64)
        av1 = convt(S2nU, wn00, wn01, wn10, wn11, I1, O1, M1, vm1_64,
                    val1_64)
        asm_a(A1S, av1, O1, M1, vm1_64)
        return carry

    lax.fori_loop(0, T_, step, 0)

    half_step0()
    half_step1()

    e_m1 = S2M[0:R0, :].reshape(R0 // 2, 2, 256)[:, 0, :]
    c0 = S2U[0:R0, 0:256].reshape(R0 // 2, 2, 256)
    e_c0, o_c0 = c0[:, 0, :], c0[:, 1, :]
    acc = None
    for di in range(3):
        for dj in range(3):
            src = (e_m1, e_c0, o_c0)[dj]
            b0 = 16 + 8 * (di - 1)
            d = jnp.dot(src[b0:b0 + M1, :], wds[di * 3 + dj],
                        preferred_element_type=F32)
            acc = d if acc is None else acc + d
    dsv = _aff_clip(acc, af_ds)
    dsc = dsv.reshape(8, 16, 256)[:, 0:8, :].reshape(64, 256)
    z1c = S2nU[O1:O1 + M1, 0:256].astype(F32).reshape(8, 16, 256)
    z1c = z1c[:, 0:8, :].reshape(64, 256)
    z = dsc + z1c
    zq = _aff_clip(jnp.dot(z, winc[...], preferred_element_type=F32), af_inc)
    out[0, :, :] = zq


def _cls_kernel(z, w, bias, out):
    out[...] = jnp.dot(z[...], w[...], preferred_element_type=F32) + bias[...]


def _tap9(w):
    return jnp.transpose(w, (2, 3, 1, 0)).reshape(9, w.shape[1], w.shape[0])


def _pack3(w):
    t = _tap9(w)
    return jnp.stack([jnp.concatenate([t[d * 3], t[d * 3 + 1], t[d * 3 + 2]],
                                      0) for d in range(3)]).astype(BF16)


def _packt(w):
    wf = jnp.flip(w, (2, 3)).transpose(1, 0, 2, 3)
    t = _tap9(wf)
    w00 = t[4]
    w01 = jnp.concatenate([t[3], t[5]], 0)
    w10 = jnp.concatenate([t[1], t[7]], 0)
    w11 = jnp.concatenate([t[0], t[2], t[6], t[8]], 0)
    return [w.astype(BF16) for w in (w00, w01, w10, w11)]


def _aff(p, bkey, bnkey):
    g = p[bnkey]['g']
    bb = p[bnkey]['b']
    bias = p[bkey] if bkey is not None else jnp.zeros_like(bb)
    return jnp.stack([g, bias * g + bb])


def kernel(x, params):
    p = params
    B = x.shape[0]

    xp = jnp.pad(x, ((0, 0), (0, 0), (1, 1), (1, 1)))
    pats = jnp.stack([xp[:, :, di:di + 64, dj:dj + 64]
                      for di in range(3) for dj in range(3)], axis=1)
    pats = pats.transpose(0, 3, 4, 1, 2).reshape(B, 64, 64, 27)
    pats = jnp.pad(pats, ((0, 0), (1, 1), (1, 1), (0, 5)))
    x9 = pats.reshape(B, RH, 32)

    w27 = jnp.transpose(p['di_w1'], (2, 3, 1, 0)).reshape(27, 64)
    w27 = jnp.concatenate([w27, jnp.zeros((5, 64), F32)], 0)

    weights = [
        w27, _aff(p, 'di_b1', 'di_bn1'),
        _tap9(p['di_w2']), _aff(p, 'di_b2', 'di_bn2'),
        _tap9(p['wx_w']), _aff(p, 'wx_b', 'wx_bn'),
        _tap9(p['wxn_w']), _aff(p, 'wxn_b', 'wxn_bn'),
        _pack3(p['s1_w']), _aff(p, 's1_b', 's1_bn'),
        _pack3(p['s2_w']), _aff(p, 's2_b', 's2_bn'),
        _pack3(p['s1n_w']), _aff(p, 's1n_b', 's1n_bn'),
        _pack3(p['s2n_w']), _aff(p, 's2n_b', 's2n_bn'),
        *_packt(p['tr_w']), *_packt(p['trn_w']),
        _tap9(p['ds_w']).astype(BF16), _aff(p, 'ds_b', 'ds_bn'),
        p['inc_w'][:, :, 0, 0].T, _aff(p, None, 'inc_bn'),
    ]

    scratch = [
        pltpu.VMEM((REO, 64), F32),
        pltpu.VMEM((REO, 64), F32),
        pltpu.VMEM((RH, 64), F32),
        pltpu.VMEM((RHN, 64), F32),
        pltpu.VMEM((4, R0, 192), BF16),
        pltpu.VMEM((2, R0, 384), BF16),
        pltpu.VMEM((R0, 512), BF16),
        pltpu.VMEM((R0, 256), BF16),
        pltpu.VMEM((4, R1, 192), BF16),
        pltpu.VMEM((2, R1, 384), BF16),
        pltpu.VMEM((R1, 512), BF16),
        pltpu.VMEM((4, M0, 64), F32),
        pltpu.VMEM((4, M1, 64), F32),
    ]
    half = B // 2
    wspecs = [pl.BlockSpec(w.shape, lambda c, i, nd=w.ndim: (0,) * nd)
              for w in weights]
    zmap = pl.pallas_call(
        _main_kernel,
        grid=(2, half),
        in_specs=[pl.BlockSpec((1, RH, 32),
                               lambda c, i: (c * half + i, 0, 0))] + wspecs,
        out_specs=pl.BlockSpec((1, 64, 256), lambda c, i: (c * half + i, 0, 0)),
        out_shape=jax.ShapeDtypeStruct((B, 64, 256), F32),
        scratch_shapes=scratch,
        compiler_params=pltpu.CompilerParams(
            dimension_semantics=("parallel", "arbitrary"),
            vmem_limit_bytes=100 * 1024 * 1024,
        ),
    )(x9, *weights)

    zflat = zmap.reshape(B, 64 * 256)
    wc = p['cls_w'].reshape(100, 256, 64).transpose(2, 1, 0).reshape(16384, 100)
    logits = pl.pallas_call(
        _cls_kernel,
        grid=(2,),
        in_specs=[
            pl.BlockSpec((B // 2, 16384), lambda i: (i, 0)),
            pl.BlockSpec((16384, 100), lambda i: (0, 0)),
            pl.BlockSpec((1, 100), lambda i: (0, 0)),
        ],
        out_specs=pl.BlockSpec((B // 2, 100), lambda i: (i, 0)),
        out_shape=jax.ShapeDtypeStruct((B, 100), F32),
        compiler_params=pltpu.CompilerParams(
            dimension_semantics=("parallel",),
        ),
    )(zflat, wc, p['cls_b'][None, :])
    return logits
